# baseline (speedup 1.0000x reference)
import jax
import jax.numpy as jnp
import numpy as np
from jax.experimental import pallas as pl

N = 10000
E = 160000
D = 256
H = 8
DH = 32
L = 6
G = 64
AIN = 92
FC = 512
BINS = 256


def _ln(x, g, b):
    m = jnp.mean(x, axis=-1, keepdims=True)
    v = jnp.mean((x - m) ** 2, axis=-1, keepdims=True)
    return (x - m) / jnp.sqrt(v + 1e-5) * g + b


def _silu(x):
    return x * jax.nn.sigmoid(x)


def _rbf(d):
    centers = jnp.linspace(0.0, 8.0, BINS)
    gamma = 1.0 / (centers[1] - centers[0]) ** 2
    return jnp.exp(-gamma * (d[:, None] - centers[None, :]) ** 2)


def kernel(x, edge_attr, edge_index, batch, ae_w1, ae_b1, ae_g1, ae_be1, ae_w2, ae_b2, rbf_w1, rbf_b1, rbf_g1, rbf_be1, rbf_w2, rbf_b2, rbf_g2, rbf_be2, rbf_w3, rbf_b3, Wq, bq, Wk, bk, Wv, bv, We, bE, ln_g, ln_b, fc_w1, fc_b1, fc_g1, fc_be1, fco_w, fco_b):
    src = edge_index[0]
    dst = edge_index[1]
    h = x @ ae_w1 + ae_b1
    h = _silu(_ln(h, ae_g1, ae_be1))
    h = h @ ae_w2 + ae_b2
    d = jnp.linalg.norm(edge_attr, axis=1)
    ef = _rbf(d) @ rbf_w1 + rbf_b1
    ef = _silu(_ln(ef, rbf_g1, rbf_be1))
    ef = ef @ rbf_w2 + rbf_b2
    ef = _silu(_ln(ef, rbf_g2, rbf_be2))
    ef = ef @ rbf_w3 + rbf_b3
    for i in range(L):
        q = (h @ Wq[i] + bq[i]).reshape(N, H, DH)
        k = (h @ Wk[i] + bk[i]).reshape(N, H, DH)
        v = (h @ Wv[i] + bv[i]).reshape(N, H, DH)
        e = (ef @ We[i] + bE[i]).reshape(E, H, DH)
        k_e = k[src] + e
        v_e = v[src] + e
        logits = jnp.sum(q[dst] * k_e, axis=-1) / np.sqrt(DH)
        mx = jax.ops.segment_max(logits, dst, num_segments=N)
        mx = jnp.where(jnp.isfinite(mx), mx, 0.0)
        ex = jnp.exp(logits - mx[dst])
        den = jax.ops.segment_sum(ex, dst, num_segments=N)
        alpha = ex / (den[dst] + 1e-16)
        agg = jax.ops.segment_sum(alpha[:, :, None] * v_e, dst, num_segments=N)
        h = _ln(h + agg.reshape(N, D), ln_g[i], ln_b[i])
    cnt = jax.ops.segment_sum(jnp.ones((N,), jnp.float32), batch, num_segments=G)
    hg = jax.ops.segment_sum(h, batch, num_segments=G) / jnp.maximum(cnt, 1.0)[:, None]
    hg = hg @ fc_w1 + fc_b1
    hg = _silu(_ln(hg, fc_g1, fc_be1))
    out = hg @ fco_w + fco_b
    return out


# speedup vs baseline: 1.0207x; 1.0207x over previous
import functools

import jax
import jax.numpy as jnp
import numpy as np
from jax.experimental import pallas as pl
from jax.experimental.pallas import tpu as pltpu

N = 10000
E = 160000
D = 256
H = 8
DH = 32
L = 6
G = 64
AIN = 92
FC = 512
BINS = 256

_F32 = jnp.float32


def _ln_in(x, g, b):
    m = jnp.mean(x, axis=-1, keepdims=True)
    v = jnp.mean((x - m) ** 2, axis=-1, keepdims=True)
    return (x - m) * jax.lax.rsqrt(v + 1e-5) * g + b


def _silu(x):
    return x * jax.nn.sigmoid(x)


def _node_enc_body(x_ref, w1_ref, b1_ref, g1_ref, be1_ref, w2_ref, b2_ref, o_ref):
    h = jnp.dot(x_ref[...], w1_ref[...], preferred_element_type=_F32) + b1_ref[...]
    h = _silu(_ln_in(h, g1_ref[...], be1_ref[...]))
    o_ref[...] = jnp.dot(h, w2_ref[...], preferred_element_type=_F32) + b2_ref[...]


def _node_encoder(xp, w1p, b1, g1, be1, w2, b2):
    NB = 1000
    return pl.pallas_call(
        _node_enc_body,
        grid=(N // NB,),
        in_specs=[
            pl.BlockSpec((NB, 128), lambda i: (i, 0)),
            pl.BlockSpec((128, D), lambda i: (0, 0)),
            pl.BlockSpec((1, D), lambda i: (0, 0)),
            pl.BlockSpec((1, D), lambda i: (0, 0)),
            pl.BlockSpec((1, D), lambda i: (0, 0)),
            pl.BlockSpec((D, D), lambda i: (0, 0)),
            pl.BlockSpec((1, D), lambda i: (0, 0)),
        ],
        out_specs=pl.BlockSpec((NB, D), lambda i: (i, 0)),
        out_shape=jax.ShapeDtypeStruct((N, D), _F32),
    )(xp, w1p, b1, g1, be1, w2, b2)


def _edge_enc_body(ea_ref, cen_ref, rw1, rb1, rg1, rbe1, rw2, rb2, rg2, rbe2, rw3, rb3,
                   We_ref, bE_ref, o_ref):
    ea = ea_ref[...]
    d = jnp.sqrt(jnp.sum(ea * ea, axis=1, keepdims=True))
    centers = cen_ref[...]
    gamma = np.float32(1.0 / (8.0 / (BINS - 1)) ** 2)
    r = jnp.exp(-gamma * (d - centers) ** 2)
    ef = jnp.dot(r, rw1[...], preferred_element_type=_F32) + rb1[...]
    ef = _silu(_ln_in(ef, rg1[...], rbe1[...]))
    ef = jnp.dot(ef, rw2[...], preferred_element_type=_F32) + rb2[...]
    ef = _silu(_ln_in(ef, rg2[...], rbe2[...]))
    ef = jnp.dot(ef, rw3[...], preferred_element_type=_F32) + rb3[...]
    for i in range(L):
        e = jnp.dot(ef, We_ref[i], preferred_element_type=_F32) + bE_ref[i].reshape(1, D)
        o_ref[i, 0] = e[:, :128]
        o_ref[i, 1] = e[:, 128:]


def _edge_encoder(eap, cen, rw1, rb1, rg1, rbe1, rw2, rb2, rg2, rbe2, rw3, rb3, We, bE):
    EB = 1000
    vec = lambda: pl.BlockSpec((1, D), lambda i: (0, 0))
    mat = lambda: pl.BlockSpec((D, D), lambda i: (0, 0))
    return pl.pallas_call(
        _edge_enc_body,
        grid=(E // EB,),
        in_specs=[
            pl.BlockSpec((EB, 4), lambda i: (i, 0)),
            vec(),
            mat(), vec(), vec(), vec(),
            mat(), vec(), vec(), vec(),
            mat(), vec(),
            pl.BlockSpec((L, D, D), lambda i: (0, 0, 0)),
            pl.BlockSpec((L, D), lambda i: (0, 0)),
        ],
        out_specs=pl.BlockSpec((L, 2, EB, 128), lambda i: (0, 0, i, 0)),
        out_shape=jax.ShapeDtypeStruct((L, 2, E, 128), _F32),
    )(eap, cen, rw1, rb1, rg1, rbe1, rw2, rb2, rg2, rbe2, rw3, rb3, We, bE)


def _qkv_body(h_ref, wq, bq_, wk, bk_, wv, bv_, q_ref, k_ref, v_ref):
    hb = h_ref[...]
    for (w, b, o) in ((wq, bq_, q_ref), (wk, bk_, k_ref), (wv, bv_, v_ref)):
        t = jnp.dot(hb, w[...], preferred_element_type=_F32) + b[...]
        o[0] = t[:, :128]
        o[1] = t[:, 128:]


def _qkv(h, wq, bq_, wk, bk_, wv, bv_):
    NB = 1000
    vec = lambda: pl.BlockSpec((1, D), lambda i: (0, 0))
    mat = lambda: pl.BlockSpec((D, D), lambda i: (0, 0))
    outs = pl.BlockSpec((2, NB, 128), lambda i: (0, i, 0))
    return pl.pallas_call(
        _qkv_body,
        grid=(N // NB,),
        in_specs=[pl.BlockSpec((NB, D), lambda i: (i, 0)),
                  mat(), vec(), mat(), vec(), mat(), vec()],
        out_specs=[outs, outs, outs],
        out_shape=[jax.ShapeDtypeStruct((2, N, 128), _F32)] * 3,
    )(h, wq, bq_, wk, bk_, wv, bv_)


def _post_body(h_ref, num_ref, den_ref, g_ref, b_ref, o_ref):
    den8 = den_ref[...]
    row = jax.lax.broadcasted_iota(jnp.int32, (H, D), 0)
    col = jax.lax.broadcasted_iota(jnp.int32, (H, D), 1)
    M = (col // DH == row).astype(_F32)
    den_exp = jnp.dot(den8, M, preferred_element_type=_F32)
    agg = num_ref[...] / (den_exp + 1e-16)
    o_ref[...] = _ln_in(h_ref[...] + agg, g_ref[...], b_ref[...])


def _post(h, numT, den8, g, b):
    NB = 1000
    return pl.pallas_call(
        _post_body,
        grid=(N // NB,),
        in_specs=[
            pl.BlockSpec((NB, D), lambda i: (i, 0)),
            pl.BlockSpec((NB, D), lambda i: (i, 0)),
            pl.BlockSpec((NB, H), lambda i: (i, 0)),
            pl.BlockSpec((1, D), lambda i: (0, 0)),
            pl.BlockSpec((1, D), lambda i: (0, 0)),
        ],
        out_specs=pl.BlockSpec((NB, D), lambda i: (i, 0)),
        out_shape=jax.ShapeDtypeStruct((N, D), _F32),
    )(h, numT, den8, g, b)


def _pool_body(h_ref, b_ref, hg_ref, cnt_ref):
    i = pl.program_id(0)
    bt = b_ref[...]
    oh = (bt == jax.lax.broadcasted_iota(jnp.int32, (1, G), 1)).astype(_F32)
    part = jax.lax.dot_general(oh, h_ref[...], (((0,), (0,)), ((), ())),
                               preferred_element_type=_F32)
    ones = jnp.ones((oh.shape[0], 1), _F32)
    pc = jax.lax.dot_general(oh, ones, (((0,), (0,)), ((), ())),
                             preferred_element_type=_F32)

    @pl.when(i == 0)
    def _():
        hg_ref[...] = jnp.zeros_like(hg_ref)
        cnt_ref[...] = jnp.zeros_like(cnt_ref)

    hg_ref[...] += part
    cnt_ref[...] += pc


def _pool(h, batch2d):
    NB = 1000
    return pl.pallas_call(
        _pool_body,
        grid=(N // NB,),
        in_specs=[
            pl.BlockSpec((NB, D), lambda i: (i, 0)),
            pl.BlockSpec((NB, 1), lambda i: (i, 0)),
        ],
        out_specs=[pl.BlockSpec((G, D), lambda i: (0, 0)),
                   pl.BlockSpec((G, 1), lambda i: (0, 0))],
        out_shape=[jax.ShapeDtypeStruct((G, D), _F32),
                   jax.ShapeDtypeStruct((G, 1), _F32)],
    )(h, batch2d)


def _head_body(hg_ref, cnt_ref, fw1, fb1, fg1, fbe1, fwo, fbo, o_ref):
    cnt = jnp.maximum(cnt_ref[...], 1.0)
    hg = hg_ref[...] / cnt
    t = jnp.dot(hg, fw1[...], preferred_element_type=_F32) + fb1[...]
    t = _silu(_ln_in(t, fg1[...], fbe1[...]))
    o_ref[...] = jnp.dot(t, fwo[...], preferred_element_type=_F32) + fbo[...]


def _head(hg, cnt, fw1, fb1, fg1, fbe1, fwo, fbo):
    return pl.pallas_call(
        _head_body,
        out_shape=jax.ShapeDtypeStruct((G, 1), _F32),
    )(hg, cnt, fw1, fb1, fg1, fbe1, fwo, fbo)


def _message_pass_jax(qtab, ktab, vtab, etab_i, src, dst):
    q = jnp.concatenate([qtab[0], qtab[1]], axis=-1).reshape(N, H, DH)
    k = jnp.concatenate([ktab[0], ktab[1]], axis=-1).reshape(N, H, DH)
    v = jnp.concatenate([vtab[0], vtab[1]], axis=-1).reshape(N, H, DH)
    e = jnp.concatenate([etab_i[0], etab_i[1]], axis=-1).reshape(E, H, DH)
    k_e = k[src] + e
    v_e = v[src] + e
    logits = jnp.sum(q[dst] * k_e, axis=-1) / np.sqrt(DH)
    w = jnp.exp(logits)
    den = jax.ops.segment_sum(w, dst, num_segments=N)
    num = jax.ops.segment_sum(w[:, :, None] * v_e, dst, num_segments=N)
    return num.reshape(N, D), den


def kernel(x, edge_attr, edge_index, batch, ae_w1, ae_b1, ae_g1, ae_be1, ae_w2, ae_b2, rbf_w1, rbf_b1, rbf_g1, rbf_be1, rbf_w2, rbf_b2, rbf_g2, rbf_be2, rbf_w3, rbf_b3, Wq, bq, Wk, bk, Wv, bv, We, bE, ln_g, ln_b, fc_w1, fc_b1, fc_g1, fc_be1, fco_w, fco_b):
    src = edge_index[0]
    dst = edge_index[1]
    r2 = lambda a: a.reshape(1, -1)

    xp = jnp.pad(x, ((0, 0), (0, 128 - AIN)))
    w1p = jnp.pad(ae_w1, ((0, 128 - AIN), (0, 0)))
    h = _node_encoder(xp, w1p, r2(ae_b1), r2(ae_g1), r2(ae_be1), ae_w2, r2(ae_b2))

    eap = jnp.pad(edge_attr, ((0, 0), (0, 1)))
    cen = jnp.asarray(np.linspace(0.0, 8.0, BINS, dtype=np.float32)).reshape(1, BINS)
    etab = _edge_encoder(eap, cen, rbf_w1, r2(rbf_b1), r2(rbf_g1), r2(rbf_be1),
                         rbf_w2, r2(rbf_b2), r2(rbf_g2), r2(rbf_be2),
                         rbf_w3, r2(rbf_b3), We, bE)

    for i in range(L):
        qtab, ktab, vtab = _qkv(h, Wq[i], r2(bq[i]), Wk[i], r2(bk[i]), Wv[i], r2(bv[i]))
        numT, den8 = _message_pass_jax(qtab, ktab, vtab, etab[i], src, dst)
        h = _post(h, numT, den8, r2(ln_g[i]), r2(ln_b[i]))

    hg, cnt = _pool(h, batch.reshape(N, 1).astype(jnp.int32))
    return _head(hg, cnt, fc_w1, r2(fc_b1), r2(fc_g1), r2(fc_be1), fco_w, r2(fco_b))
